# baseline (speedup 1.0000x reference)
import jax
import jax.numpy as jnp
from jax.experimental import pallas as pl

_MAX_LEN = 200
_EMBED_DIM = 64
_FLAT = _MAX_LEN * _EMBED_DIM
_BB = 256


def _broadcast_body(pe_ref, o_ref):
    o_ref[...] = jnp.broadcast_to(pe_ref[...], o_ref.shape)


def kernel(x, pe_weight):
    batch = x.shape[0]
    pe_flat = pe_weight.reshape(1, _FLAT)
    out = pl.pallas_call(
        _broadcast_body,
        grid=(batch // _BB,),
        in_specs=[pl.BlockSpec((1, _FLAT), lambda i: (0, 0))],
        out_specs=pl.BlockSpec((_BB, _FLAT), lambda i: (i, 0)),
        out_shape=jax.ShapeDtypeStruct((batch, _FLAT), jnp.float32),
    )(pe_flat)
    return out.reshape(batch, _MAX_LEN, _EMBED_DIM)


# speedup vs baseline: 1.0018x; 1.0018x over previous
import jax
import jax.numpy as jnp
from jax.experimental import pallas as pl
from jax.experimental.pallas import tpu as pltpu

_MAX_LEN = 200
_EMBED_DIM = 64
_FLAT = _MAX_LEN * _EMBED_DIM
_BB = 256
_NCHUNK = 4096 // _BB


def _body(pe_ref, o_hbm, scratch, sems):
    scratch[...] = jnp.broadcast_to(pe_ref[...], scratch.shape)
    copies = [
        pltpu.make_async_copy(
            scratch, o_hbm.at[pl.ds(i * _BB, _BB), :], sems.at[i]
        )
        for i in range(_NCHUNK)
    ]
    for c in copies:
        c.start()
    for c in copies:
        c.wait()


def kernel(x, pe_weight):
    batch = x.shape[0]
    pe_flat = pe_weight.reshape(1, _FLAT)
    out = pl.pallas_call(
        _body,
        in_specs=[pl.BlockSpec(memory_space=pltpu.MemorySpace.VMEM)],
        out_specs=pl.BlockSpec(memory_space=pltpu.MemorySpace.HBM),
        out_shape=jax.ShapeDtypeStruct((batch, _FLAT), jnp.float32),
        scratch_shapes=[
            pltpu.VMEM((_BB, _FLAT), jnp.float32),
            pltpu.SemaphoreType.DMA((_NCHUNK,)),
        ],
    )(pe_flat)
    return out.reshape(batch, _MAX_LEN, _EMBED_DIM)
